# baseline (speedup 1.0000x reference)
import functools
import math

import jax
import jax.numpy as jnp
from jax import lax
from jax.experimental import pallas as pl
from jax.experimental.pallas import tpu as pltpu
from jax.experimental.pallas import tpu_sc as plsc

_E = 8
_K = 2
_INV_SQRT2 = 1.0 / math.sqrt(2.0)
_B = 1024
_BS = 1024
_NW = 32
_SUB = 128


def _router_block(z_ref, wrT_ref, br_ref, tril_ref,
                  probs_ref, idx_ref, zf_ref, idxT_ref, rankT_ref, wT_ref,
                  imp_ref, load_ref, offs_ref, base_scr):
    B = z_ref.shape[0]
    Dh = zf_ref.shape[1]
    zb = z_ref[...]

    lo = jax.lax.bitcast_convert_type(
        zb[:, :Dh].astype(jnp.bfloat16).astype(jnp.float32), jnp.uint32)
    hi = jax.lax.bitcast_convert_type(
        zb[:, Dh:].astype(jnp.bfloat16).astype(jnp.float32), jnp.uint32)
    word = jnp.bitwise_or(
        jax.lax.shift_right_logical(lo, jnp.uint32(16)), hi)
    zf_ref[...] = jax.lax.bitcast_convert_type(word, jnp.float32)

    logits = jnp.dot(zb, wrT_ref[...], preferred_element_type=jnp.float32)
    logits = logits + br_ref[...]
    m = jnp.max(logits, axis=-1, keepdims=True)
    ex = jnp.exp(logits - m)
    probs = ex / jnp.sum(ex, axis=-1, keepdims=True)

    i1 = jnp.argmax(probs, axis=-1)
    p1 = jnp.max(probs, axis=-1)
    eids = jax.lax.broadcasted_iota(jnp.int32, (B, _E), 1)
    probs_m = jnp.where(eids == i1[:, None], -1.0, probs)
    i2 = jnp.argmax(probs_m, axis=-1)
    p2 = jnp.max(probs_m, axis=-1)
    denom = jnp.maximum(p1 + p2, 1e-8)

    probs_ref[...] = probs
    idx_ref[...] = jnp.stack([i1, i2], axis=1).astype(jnp.int32)
    idxT_ref[...] = jnp.stack([i1, i2], axis=0).astype(jnp.int32)
    wT_ref[...] = jnp.stack([p1 / denom, p2 / denom], axis=0)

    oh1 = (eids == i1[:, None]).astype(jnp.float32)
    oh2 = (eids == i2[:, None]).astype(jnp.float32)
    ohsum = oh1 + oh2

    @pl.when(pl.program_id(0) == 0)
    def _init():
        imp_ref[...] = jnp.zeros_like(imp_ref)
        load_ref[...] = jnp.zeros_like(load_ref)
        base_scr[...] = jnp.zeros_like(base_scr)

    ranks_strict = jnp.dot(tril_ref[...], ohsum.astype(jnp.bfloat16),
                           preferred_element_type=jnp.float32)
    ranks_abs = ranks_strict + base_scr[...]
    rank1 = jnp.sum(ranks_abs * oh1, axis=-1)
    rank2 = jnp.sum(ranks_abs * oh2, axis=-1)
    rankT_ref[...] = jnp.stack([rank1, rank2], axis=0).astype(jnp.int32)

    colsum = jnp.sum(ohsum, axis=0, keepdims=True)
    base_scr[...] += colsum
    load_ref[...] += colsum
    imp_ref[...] += jnp.sum(probs, axis=0, keepdims=True)

    @pl.when(pl.program_id(0) == pl.num_programs(0) - 1)
    def _fin():
        cnts = base_scr[...]
        padded = jnp.floor((cnts + (_BS - 1)) * (1.0 / _BS)) * _BS
        ii = jax.lax.broadcasted_iota(jnp.int32, (_E, _E), 0)
        jj = jax.lax.broadcasted_iota(jnp.int32, (_E, _E), 1)
        ustrict = (ii < jj).astype(jnp.float32)
        offs8 = jnp.dot(padded, ustrict,
                        preferred_element_type=jnp.float32)
        offs_ref[...] = jnp.concatenate(
            [offs8, jnp.zeros_like(offs8)], axis=-1).astype(jnp.int32)


def _router_call(z, wrT, br2, tril):
    N, D = z.shape
    nblk = N // _B
    full = lambda *shape: pl.BlockSpec(shape, lambda i: (0,) * len(shape))
    out_shapes = (
        jax.ShapeDtypeStruct((N, _E), jnp.float32),
        jax.ShapeDtypeStruct((N, _K), jnp.int32),
        jax.ShapeDtypeStruct((N, D // 2), jnp.float32),
        jax.ShapeDtypeStruct((_K, N), jnp.int32),
        jax.ShapeDtypeStruct((_K, N), jnp.int32),
        jax.ShapeDtypeStruct((_K, N), jnp.float32),
        jax.ShapeDtypeStruct((1, _E), jnp.float32),
        jax.ShapeDtypeStruct((1, _E), jnp.float32),
        jax.ShapeDtypeStruct((1, 16), jnp.int32),
    )
    return pl.pallas_call(
        _router_block,
        grid=(nblk,),
        in_specs=[
            pl.BlockSpec((_B, D), lambda i: (i, 0)),
            full(D, _E),
            full(1, _E),
            full(_B, _B),
        ],
        out_specs=(
            pl.BlockSpec((_B, _E), lambda i: (i, 0)),
            pl.BlockSpec((_B, _K), lambda i: (i, 0)),
            pl.BlockSpec((_B, D // 2), lambda i: (i, 0)),
            pl.BlockSpec((_K, _B), lambda i: (0, i)),
            pl.BlockSpec((_K, _B), lambda i: (0, i)),
            pl.BlockSpec((_K, _B), lambda i: (0, i)),
            pl.BlockSpec((1, _E), lambda i: (0, 0)),
            pl.BlockSpec((1, _E), lambda i: (0, 0)),
            pl.BlockSpec((1, 16), lambda i: (0, 0)),
        ),
        out_shape=out_shapes,
        scratch_shapes=[pltpu.VMEM((1, _E), jnp.float32)],
        compiler_params=pltpu.CompilerParams(
            dimension_semantics=("arbitrary",),
        ),
    )(z, wrT, br2, tril)


def _dispatch_sc(zf, idxT, rankT, offs16):
    N, D = zf.shape
    S = N * _K + _E * _BS
    CT = N // _NW
    NSUB = CT // _SUB
    NR = N // _SUB
    mesh = plsc.VectorSubcoreMesh(core_axis_name="c", subcore_axis_name="s")

    @functools.partial(
        pl.kernel,
        out_type=(
            jax.ShapeDtypeStruct((S, D), jnp.float32),
            jax.ShapeDtypeStruct((_K, NR, _SUB), jnp.int32),
        ),
        mesh=mesh,
        scratch_types=[
            pltpu.VMEM((16,), jnp.int32),
            pltpu.VMEM((CT // _SUB, _SUB), jnp.int32),
            pltpu.VMEM((CT // _SUB, _SUB), jnp.int32),
            pltpu.VMEM((2 * NSUB, _SUB), jnp.int32),
            pltpu.VMEM((_SUB, D), jnp.float32),
            pltpu.VMEM((_SUB, D), jnp.float32),
            pltpu.SemaphoreType.DMA,
            pltpu.SemaphoreType.DMA,
        ],
        compiler_params=pltpu.CompilerParams(needs_layout_passes=False),
    )
    def k(zf_hbm, idxT_hbm, rankT_hbm, offs_hbm,
          zg_hbm, destT_hbm,
          offs_v, idx_v, rank_v, dest_v, rows_a, rows_b,
          sem_ld, sem_sc):
        wid = lax.axis_index("s") * 2 + lax.axis_index("c")
        base = wid * CT
        b128 = wid * NSUB

        pltpu.sync_copy(offs_hbm, offs_v)

        for kk in range(_K):
            pltpu.sync_copy(idxT_hbm.at[kk, pl.ds(b128, NSUB)], idx_v)
            pltpu.sync_copy(rankT_hbm.at[kk, pl.ds(b128, NSUB)], rank_v)
            for j in range(NSUB):
                def body(i8, _, j=j, kk=kk):
                    sl = pl.ds(i8 * 16, 16)
                    iv = idx_v[j, sl]
                    rv = rank_v[j, sl]
                    dv = plsc.load_gather(offs_v, [iv]) + rv
                    dest_v[kk * NSUB + j, sl] = dv
                    return 0
                lax.fori_loop(0, _SUB // 16, body, 0)
            pltpu.sync_copy(dest_v.at[pl.ds(kk * NSUB, NSUB)],
                            destT_hbm.at[kk, pl.ds(b128, NSUB)])

        bufs = (rows_a, rows_b)
        d = pltpu.async_copy(zf_hbm.at[pl.ds(base, _SUB)], rows_a, sem_ld)
        for j in range(NSUB):
            d.wait()
            if j + 1 < NSUB:
                d = pltpu.async_copy(
                    zf_hbm.at[pl.ds(base + (j + 1) * _SUB, _SUB)],
                    bufs[(j + 1) % 2], sem_ld)
            cur = bufs[j % 2]
            s0 = pltpu.async_copy(cur, zg_hbm.at[dest_v.at[j]], sem_sc)
            s1 = pltpu.async_copy(cur, zg_hbm.at[dest_v.at[NSUB + j]],
                                  sem_sc)
            s0.wait()
            s1.wait()

    idxT3 = idxT.reshape(_K, NR, _SUB)
    rankT3 = rankT.reshape(_K, NR, _SUB)
    return k(zf, idxT3, rankT3, offs16)


def _gemm_block(offs_sref, zg_ref, w1Ta_ref, w1Tb_ref, wpTa_ref, wpTb_ref,
                b1_ref, w2T_ref, b2_ref, u_ref, su_ref, s0_ref, ys_ref):
    H = b1_ref.shape[2]
    zi = jax.lax.bitcast_convert_type(zg_ref[...], jnp.int32)
    za = jax.lax.bitcast_convert_type(
        jax.lax.shift_left(zi, 16), jnp.float32).astype(jnp.bfloat16)
    zbh = jax.lax.bitcast_convert_type(
        jnp.bitwise_and(zi, jnp.int32(-65536)), jnp.float32
    ).astype(jnp.bfloat16)
    hpre = (jnp.dot(za, w1Ta_ref[0], preferred_element_type=jnp.float32)
            + jnp.dot(zbh, w1Tb_ref[0], preferred_element_type=jnp.float32))
    hpre = hpre + b1_ref[0]
    xp = (jnp.dot(za, wpTa_ref[0], preferred_element_type=jnp.float32)
          + jnp.dot(zbh, wpTb_ref[0], preferred_element_type=jnp.float32))
    h = 0.5 * hpre * (1.0 + jax.lax.erf(hpre * _INV_SQRT2))
    h2 = jnp.dot(h.astype(jnp.bfloat16), w2T_ref[0],
                 preferred_element_type=jnp.float32)
    r = h2 + b2_ref[0] + xp
    s1 = jnp.sum(r, axis=-1)
    s2 = jnp.sum(r * r, axis=-1)
    sru = jnp.sum(r * u_ref[0], axis=-1)
    mu = s1 * (1.0 / H)
    var = s2 * (1.0 / H) - mu * mu
    inv = jax.lax.rsqrt(var + 1e-5)
    ys_ref[...] = (inv * (sru - mu * su_ref[0, 0, 0]) + s0_ref[0, 0, 0])[None, None, :]


def _gemm_call(offs16, zg, w1Ta, w1Tb, wpTa, wpTb, b1, w2T, b2, u, su_c, s0_c):
    S, D2 = zg.shape
    E, H = b1.shape
    b1 = b1.reshape(E, 1, H)
    b2 = b2.reshape(E, 1, H)
    u = u.reshape(E, 1, H)
    su_c = su_c.reshape(E, 1, 1)
    s0_c = s0_c.reshape(E, 1, 1)
    nblk = S // _BS

    def emap(j, offs):
        e = jnp.int32(0)
        for t in range(1, _E):
            e = e + (j * _BS >= offs[t]).astype(jnp.int32)
        return (e, 0, 0)

    def emap3(j, offs):
        e, _, _ = emap(j, offs)
        return (e, 0, 0)

    grid_spec = pltpu.PrefetchScalarGridSpec(
        num_scalar_prefetch=1,
        grid=(nblk,),
        in_specs=[
            pl.BlockSpec((_BS, D2), lambda j, offs: (j, 0)),
            pl.BlockSpec((1, D2, H), emap),
            pl.BlockSpec((1, D2, H), emap),
            pl.BlockSpec((1, D2, H), emap),
            pl.BlockSpec((1, D2, H), emap),
            pl.BlockSpec((1, 1, H), emap3),
            pl.BlockSpec((1, H, H), emap),
            pl.BlockSpec((1, 1, H), emap3),
            pl.BlockSpec((1, 1, H), emap3),
            pl.BlockSpec((1, 1, 1), emap3),
            pl.BlockSpec((1, 1, 1), emap3),
        ],
        out_specs=pl.BlockSpec((1, 1, _BS), lambda j, offs: (j, 0, 0)),
    )
    return pl.pallas_call(
        _gemm_block,
        grid_spec=grid_spec,
        out_shape=jax.ShapeDtypeStruct((nblk, 1, _BS), jnp.float32),
        compiler_params=pltpu.CompilerParams(
            dimension_semantics=("arbitrary",),
        ),
    )(offs16, zg, w1Ta, w1Tb, wpTa, wpTb, b1, w2T, b2, u, su_c, s0_c)


def _combine_sc(ys, destT, wT):
    S = ys.shape[0]
    NR = destT.shape[1]
    N = NR * _SUB
    NSUB = NR // _NW
    mesh = plsc.VectorSubcoreMesh(core_axis_name="c", subcore_axis_name="s")

    @functools.partial(
        pl.kernel,
        out_type=jax.ShapeDtypeStruct((NR, _SUB), jnp.float32),
        mesh=mesh,
        scratch_types=[
            pltpu.VMEM((2 * NSUB, _SUB), jnp.int32),
            pltpu.VMEM((2 * NSUB, _SUB), jnp.float32),
            pltpu.VMEM((2 * NSUB, _SUB), jnp.float32),
            pltpu.VMEM((NSUB, _SUB), jnp.float32),
            pltpu.SemaphoreType.DMA,
        ],
        compiler_params=pltpu.CompilerParams(needs_layout_passes=False),
    )
    def k(ys_hbm, destT_hbm, wT_hbm, yhat_hbm,
          dest_v, vals_v, w_v, y_v, sem):
        wid = lax.axis_index("s") * 2 + lax.axis_index("c")
        b128 = wid * NSUB
        pltpu.sync_copy(destT_hbm.at[0, pl.ds(b128, NSUB)],
                        dest_v.at[pl.ds(0, NSUB)])
        pltpu.sync_copy(destT_hbm.at[1, pl.ds(b128, NSUB)],
                        dest_v.at[pl.ds(NSUB, NSUB)])
        pltpu.sync_copy(wT_hbm.at[0, pl.ds(b128, NSUB)],
                        w_v.at[pl.ds(0, NSUB)])
        pltpu.sync_copy(wT_hbm.at[1, pl.ds(b128, NSUB)],
                        w_v.at[pl.ds(NSUB, NSUB)])
        cps = [pltpu.async_copy(ys_hbm.at[dest_v.at[j]], vals_v.at[j], sem)
               for j in range(2 * NSUB)]
        for cp in cps:
            cp.wait()
        for j in range(NSUB):
            def body(i, _, j=j):
                sl = pl.ds(i * 16, 16)
                y_v[j, sl] = (w_v[j, sl] * vals_v[j, sl]
                              + w_v[NSUB + j, sl] * vals_v[NSUB + j, sl])
                return 0
            lax.fori_loop(0, _SUB // 16, body, 0)
        pltpu.sync_copy(y_v, yhat_hbm.at[pl.ds(b128, NSUB)])

    return k(ys, destT, wT)


def kernel(z, Wr, br, W1, b1, W2, b2, Wproj, ln_g, ln_b, Wo, bo):
    N, D = z.shape
    E, H = b1.shape
    Dh = D // 2

    w1T = W1.transpose(0, 2, 1).astype(jnp.bfloat16)
    wpT = Wproj.transpose(0, 2, 1).astype(jnp.bfloat16)
    w1Ta, w1Tb = w1T[:, :Dh, :], w1T[:, Dh:, :]
    wpTa, wpTb = wpT[:, :Dh, :], wpT[:, Dh:, :]
    w2T = W2.transpose(0, 2, 1).astype(jnp.bfloat16)
    wrT = Wr.T
    wo = Wo[:, 0, :]
    u = ln_g * wo
    su_c = jnp.sum(u, axis=1).reshape(E, 1)
    s0_c = (jnp.sum(ln_b * wo, axis=1) + bo[:, 0]).reshape(E, 1)
    tril = jnp.tril(jnp.ones((_B, _B), jnp.bfloat16), k=-1)

    (probs, topk_idx, zf, idxT, rankT, wT2, imp, loadsum,
     offs2d) = _router_call(z, wrT, br.reshape(1, E), tril)

    offs16 = offs2d.reshape(16)
    wT = wT2.reshape(_K, N // _SUB, _SUB)

    zg, destT = _dispatch_sc(zf, idxT, rankT, offs16)
    ys = _gemm_call(offs16, zg, w1Ta, w1Tb, wpTa, wpTb,
                    b1, w2T, b2, u, su_c, s0_c)
    y_hat = _combine_sc(ys.reshape(-1), destT, wT)

    inv_n = 1.0 / N
    return (y_hat.reshape(N, 1), probs, topk_idx,
            imp[0] * inv_n, loadsum[0] * inv_n)
